# baseline (speedup 1.0000x reference)
import functools

import jax
import jax.numpy as jnp
from jax import lax
from jax.experimental import pallas as pl
from jax.experimental.pallas import tpu as pltpu
from jax.experimental.pallas import tpu_sc as plsc

BATCH = 16384
HIDDEN = 64
LANES = 16
N_CHUNKS = 4
CHUNK = 128
B_PER_W = N_CHUNKS * CHUNK
N_WORKERS = BATCH // B_PER_W
NODE_BLK = 4096
NODE_SPLIT = 123 * NODE_BLK
REL_SPLIT = 256
RED_BLK = 2048


def _pack_body(a_ref, b_ref, dst_ref):
    i = pl.program_id(0)
    n = pl.num_programs(0)

    @pl.when(i < n - 1)
    def _():
        dst_ref[...] = jnp.concatenate([a_ref[...].T, b_ref[...].T], axis=1)

    @pl.when(i == n - 1)
    def _():
        dst_ref[...] = jnp.zeros_like(dst_ref)


def _gather_table(table_t, split, blk):
    nblk = split // blk
    last = pl.cdiv(table_t.shape[1], blk) - 1
    return pl.pallas_call(
        _pack_body,
        grid=(nblk + 1,),
        in_specs=[
            pl.BlockSpec((HIDDEN, blk),
                         lambda i, last=last: (0, jnp.minimum(i, last))),
            pl.BlockSpec((HIDDEN, blk),
                         lambda i, nblk=nblk, last=last:
                         (0, jnp.minimum(i + nblk, last))),
        ],
        out_specs=pl.BlockSpec((blk, 2 * HIDDEN), lambda i: (i, 0)),
        out_shape=jax.ShapeDtypeStruct((split + blk, 2 * HIDDEN),
                                       jnp.float32),
    )(table_t, table_t)


def _reduce_body(p_ref, o_ref):
    o_ref[...] = jnp.sum(p_ref[...], axis=1)


def _distmult_sc_body(node_hbm, rel_hbm, hlo_hbm, hhi_hbm, rlo_hbm, rhi_hbm,
                      tlo_hbm, thi_hbm, out_hbm, idx_hlo, idx_hhi, idx_rlo,
                      idx_rhi, idx_tlo, idx_thi, hlo_v, hhi_v, rlo_v, rhi_v,
                      tlo_v, thi_v, pacc_v, sem_h, sem_r, sem_t):
    wid = lax.axis_index("s") * 2 + lax.axis_index("c")

    pltpu.sync_copy(hlo_hbm.at[wid], idx_hlo)
    pltpu.sync_copy(hhi_hbm.at[wid], idx_hhi)
    pltpu.sync_copy(rlo_hbm.at[wid], idx_rlo)
    pltpu.sync_copy(rhi_hbm.at[wid], idx_rhi)
    pltpu.sync_copy(tlo_hbm.at[wid], idx_tlo)
    pltpu.sync_copy(thi_hbm.at[wid], idx_thi)

    def gather(j):
        return (
            pltpu.make_async_copy(node_hbm.at[idx_hlo.at[j]], hlo_v, sem_h),
            pltpu.make_async_copy(node_hbm.at[idx_hhi.at[j]], hhi_v, sem_h),
            pltpu.make_async_copy(rel_hbm.at[idx_rlo.at[j]], rlo_v, sem_r),
            pltpu.make_async_copy(rel_hbm.at[idx_rhi.at[j]], rhi_v, sem_r),
            pltpu.make_async_copy(node_hbm.at[idx_tlo.at[j]], tlo_v, sem_t),
            pltpu.make_async_copy(node_hbm.at[idx_thi.at[j]], thi_v, sem_t),
        )

    for j in range(N_CHUNKS):
        for c in gather(j):
            c.start()
        for c in gather(j):
            c.wait()

        def row_body(i, _):
            acc = jnp.zeros((LANES,), jnp.float32)
            for g in range(HIDDEN // LANES):
                lo = pl.ds(g * LANES, LANES)
                hi = pl.ds(HIDDEN + g * LANES, LANES)
                hk = hlo_v[i, lo] + hhi_v[i, hi]
                rk = rlo_v[i, lo] + rhi_v[i, hi]
                tk = tlo_v[i, lo] + thi_v[i, hi]
                acc = acc + hk * rk * tk
            pacc_v[i] = acc
            return 0

        lax.fori_loop(0, CHUNK, row_body, 0)
        pltpu.sync_copy(
            pacc_v, out_hbm.at[pl.ds(wid * B_PER_W + j * CHUNK, CHUNK)])


def _split_idx(idx, split):
    in_lo = idx < split
    lo = jnp.where(in_lo, idx, split).reshape(N_WORKERS, N_CHUNKS, CHUNK)
    hi = jnp.where(in_lo, split, idx - split).reshape(
        N_WORKERS, N_CHUNKS, CHUNK)
    return lo, hi


def kernel(head_index, rel_type, tail_index, node_emb, rel_emb):
    hlo, hhi = _split_idx(head_index, NODE_SPLIT)
    tlo, thi = _split_idx(tail_index, NODE_SPLIT)
    rlo, rhi = _split_idx(rel_type, REL_SPLIT)
    node_tab = _gather_table(node_emb.T, NODE_SPLIT, NODE_BLK)
    rel_tab = _gather_table(rel_emb.T, REL_SPLIT, REL_SPLIT)

    mesh = plsc.VectorSubcoreMesh(core_axis_name="c", subcore_axis_name="s")
    idx_t = pltpu.VMEM((N_CHUNKS, CHUNK), jnp.int32)
    row_t = pltpu.VMEM((CHUNK, 2 * HIDDEN), jnp.float32)
    sc_run = functools.partial(
        pl.kernel,
        mesh=mesh,
        compiler_params=pltpu.CompilerParams(use_tc_tiling_on_sc=True),
        out_type=jax.ShapeDtypeStruct((BATCH, LANES), jnp.float32),
        scratch_types=[
            idx_t, idx_t, idx_t, idx_t, idx_t, idx_t,
            row_t, row_t, row_t, row_t, row_t, row_t,
            pltpu.VMEM((CHUNK, LANES), jnp.float32),
            pltpu.SemaphoreType.DMA,
            pltpu.SemaphoreType.DMA,
            pltpu.SemaphoreType.DMA,
        ],
    )(_distmult_sc_body)
    pacc = sc_run(node_tab, rel_tab, hlo, hhi, rlo, rhi, tlo, thi)

    return pl.pallas_call(
        _reduce_body,
        grid=(BATCH // RED_BLK,),
        in_specs=[pl.BlockSpec((RED_BLK, LANES), lambda i: (i, 0))],
        out_specs=pl.BlockSpec((RED_BLK,), lambda i: (i,)),
        out_shape=jax.ShapeDtypeStruct((BATCH,), jnp.float32),
    )(pacc)


# speedup vs baseline: 1.0027x; 1.0027x over previous
import functools

import jax
import jax.numpy as jnp
from jax import lax
from jax.experimental import pallas as pl
from jax.experimental.pallas import tpu as pltpu
from jax.experimental.pallas import tpu_sc as plsc

BATCH = 16384
HIDDEN = 64
LANES = 16
N_CHUNKS = 4
CHUNK = 128
B_PER_W = N_CHUNKS * CHUNK
N_WORKERS = BATCH // B_PER_W
NODE_BLK = 4096
NODE_SPLIT = 122 * NODE_BLK
NODE_DATA = 123 * NODE_BLK
NODE_ZERO = NODE_DATA
RED_BLK = 2048


def _pack_split_body(a_ref, b_ref, dst_ref):
    dst_ref[...] = jnp.concatenate([a_ref[...].T, b_ref[...].T], axis=1)


def _zero_body(tab_any, dst_ref):
    del tab_any
    dst_ref[...] = jnp.zeros_like(dst_ref)


def _node_table(table_t):
    s_blk = NODE_SPLIT // NODE_BLK
    tab = pl.pallas_call(
        _pack_split_body,
        grid=(NODE_DATA // NODE_BLK,),
        in_specs=[
            pl.BlockSpec((HIDDEN, NODE_BLK), lambda i: (0, i)),
            pl.BlockSpec((HIDDEN, NODE_BLK),
                         lambda i, s_blk=s_blk: (0, i + s_blk)),
        ],
        out_specs=pl.BlockSpec((NODE_BLK, 2 * HIDDEN), lambda i: (i, 0)),
        out_shape=jax.ShapeDtypeStruct((NODE_DATA + NODE_BLK, 2 * HIDDEN),
                                       jnp.float32),
    )(table_t, table_t)
    return pl.pallas_call(
        _zero_body,
        grid=(1,),
        in_specs=[pl.BlockSpec(memory_space=pltpu.MemorySpace.HBM)],
        out_specs=pl.BlockSpec(
            (NODE_BLK, 2 * HIDDEN),
            lambda i: (NODE_DATA // NODE_BLK, 0)),
        out_shape=jax.ShapeDtypeStruct((NODE_DATA + NODE_BLK, 2 * HIDDEN),
                                       jnp.float32),
        input_output_aliases={0: 0},
    )(tab)


def _pack_dup_body(a_ref, dst_ref):
    emb = a_ref[...].T
    dst_ref[:, :HIDDEN] = emb
    dst_ref[:, HIDDEN:] = emb


def _rel_table(table_t):
    return pl.pallas_call(
        _pack_dup_body,
        grid=(1,),
        in_specs=[pl.BlockSpec((HIDDEN, 512), lambda i: (0, 0))],
        out_specs=pl.BlockSpec((512, 2 * HIDDEN), lambda i: (0, 0)),
        out_shape=jax.ShapeDtypeStruct((512, 2 * HIDDEN), jnp.float32),
    )(table_t)


def _reduce_body(p_ref, o_ref):
    o_ref[...] = jnp.sum(p_ref[...], axis=1)


def _distmult_sc_body(node_hbm, rel_hbm, hlo_hbm, hhi_hbm, rel_idx_hbm,
                      tlo_hbm, thi_hbm, out_hbm, idx_hlo, idx_hhi, idx_r,
                      idx_tlo, idx_thi, hlo_v, hhi_v, r_v, tlo_v, thi_v,
                      pacc_v, sem_h, sem_r, sem_t):
    wid = lax.axis_index("s") * 2 + lax.axis_index("c")

    pltpu.sync_copy(hlo_hbm.at[wid], idx_hlo)
    pltpu.sync_copy(hhi_hbm.at[wid], idx_hhi)
    pltpu.sync_copy(rel_idx_hbm.at[wid], idx_r)
    pltpu.sync_copy(tlo_hbm.at[wid], idx_tlo)
    pltpu.sync_copy(thi_hbm.at[wid], idx_thi)

    def gather(j):
        return (
            pltpu.make_async_copy(node_hbm.at[idx_hlo.at[j]], hlo_v, sem_h),
            pltpu.make_async_copy(node_hbm.at[idx_hhi.at[j]], hhi_v, sem_h),
            pltpu.make_async_copy(rel_hbm.at[idx_r.at[j]], r_v, sem_r),
            pltpu.make_async_copy(node_hbm.at[idx_tlo.at[j]], tlo_v, sem_t),
            pltpu.make_async_copy(node_hbm.at[idx_thi.at[j]], thi_v, sem_t),
        )

    for j in range(N_CHUNKS):
        for c in gather(j):
            c.start()
        for c in gather(j):
            c.wait()

        def row_body(i, _):
            acc = jnp.zeros((LANES,), jnp.float32)
            for g in range(HIDDEN // LANES):
                lo = pl.ds(g * LANES, LANES)
                hi = pl.ds(HIDDEN + g * LANES, LANES)
                hk = hlo_v[i, lo] + hhi_v[i, hi]
                tk = tlo_v[i, lo] + thi_v[i, hi]
                acc = acc + hk * r_v[i, lo] * tk
            pacc_v[i] = acc
            return 0

        lax.fori_loop(0, CHUNK, row_body, 0)
        pltpu.sync_copy(
            pacc_v, out_hbm.at[pl.ds(wid * B_PER_W + j * CHUNK, CHUNK)])


def _split_idx(idx):
    in_lo = idx < NODE_SPLIT
    lo = jnp.where(in_lo, idx, NODE_ZERO).reshape(
        N_WORKERS, N_CHUNKS, CHUNK)
    hi = jnp.where(in_lo, NODE_ZERO, idx - NODE_SPLIT).reshape(
        N_WORKERS, N_CHUNKS, CHUNK)
    return lo, hi


def kernel(head_index, rel_type, tail_index, node_emb, rel_emb):
    hlo, hhi = _split_idx(head_index)
    tlo, thi = _split_idx(tail_index)
    rel3d = rel_type.reshape(N_WORKERS, N_CHUNKS, CHUNK)
    node_tab = _node_table(node_emb.T)
    rel_tab = _rel_table(rel_emb.T)

    mesh = plsc.VectorSubcoreMesh(core_axis_name="c", subcore_axis_name="s")
    idx_t = pltpu.VMEM((N_CHUNKS, CHUNK), jnp.int32)
    row_t = pltpu.VMEM((CHUNK, 2 * HIDDEN), jnp.float32)
    sc_run = functools.partial(
        pl.kernel,
        mesh=mesh,
        compiler_params=pltpu.CompilerParams(use_tc_tiling_on_sc=True),
        out_type=jax.ShapeDtypeStruct((BATCH, LANES), jnp.float32),
        scratch_types=[
            idx_t, idx_t, idx_t, idx_t, idx_t,
            row_t, row_t, row_t, row_t, row_t,
            pltpu.VMEM((CHUNK, LANES), jnp.float32),
            pltpu.SemaphoreType.DMA,
            pltpu.SemaphoreType.DMA,
            pltpu.SemaphoreType.DMA,
        ],
    )(_distmult_sc_body)
    pacc = sc_run(node_tab, rel_tab, hlo, hhi, rel3d, tlo, thi)

    return pl.pallas_call(
        _reduce_body,
        grid=(BATCH // RED_BLK,),
        in_specs=[pl.BlockSpec((RED_BLK, LANES), lambda i: (i, 0))],
        out_specs=pl.BlockSpec((RED_BLK,), lambda i: (i,)),
        out_shape=jax.ShapeDtypeStruct((BATCH,), jnp.float32),
    )(pacc)


# speedup vs baseline: 4.9123x; 4.8991x over previous
import functools

import jax
import jax.numpy as jnp
from jax import lax
from jax.experimental import pallas as pl
from jax.experimental.pallas import tpu as pltpu
from jax.experimental.pallas import tpu_sc as plsc

BATCH = 16384
HIDDEN = 64
LANES = 16
N_CHUNKS = 4
CHUNK = 128
B_PER_W = N_CHUNKS * CHUNK
N_WORKERS = BATCH // B_PER_W
NODE_BLK = 4096
NODE_SPLIT = 122 * NODE_BLK
NODE_DATA = 123 * NODE_BLK
NODE_ZERO = NODE_DATA
RED_BLK = 2048


def _pack_split_body(a_ref, b_ref, dst_ref):
    dst_ref[...] = jnp.concatenate([a_ref[...].T, b_ref[...].T], axis=1)


def _zero_body(tab_any, dst_ref):
    del tab_any
    dst_ref[...] = jnp.zeros_like(dst_ref)


def _node_table(table_t):
    s_blk = NODE_SPLIT // NODE_BLK
    tab = pl.pallas_call(
        _pack_split_body,
        grid=(NODE_DATA // NODE_BLK,),
        in_specs=[
            pl.BlockSpec((HIDDEN, NODE_BLK), lambda i: (0, i)),
            pl.BlockSpec((HIDDEN, NODE_BLK),
                         lambda i, s_blk=s_blk: (0, i + s_blk)),
        ],
        out_specs=pl.BlockSpec((NODE_BLK, 2 * HIDDEN), lambda i: (i, 0)),
        out_shape=jax.ShapeDtypeStruct((NODE_DATA + NODE_BLK, 2 * HIDDEN),
                                       jnp.float32),
    )(table_t, table_t)
    return pl.pallas_call(
        _zero_body,
        grid=(1,),
        in_specs=[pl.BlockSpec(memory_space=pltpu.MemorySpace.HBM)],
        out_specs=pl.BlockSpec(
            (NODE_BLK, 2 * HIDDEN),
            lambda i: (NODE_DATA // NODE_BLK, 0)),
        out_shape=jax.ShapeDtypeStruct((NODE_DATA + NODE_BLK, 2 * HIDDEN),
                                       jnp.float32),
        input_output_aliases={0: 0},
    )(tab)


def _pack_dup_body(a_ref, dst_ref):
    emb = a_ref[...].T
    dst_ref[:, :HIDDEN] = emb
    dst_ref[:, HIDDEN:] = emb


def _rel_table(table_t):
    return pl.pallas_call(
        _pack_dup_body,
        grid=(1,),
        in_specs=[pl.BlockSpec((HIDDEN, 512), lambda i: (0, 0))],
        out_specs=pl.BlockSpec((512, 2 * HIDDEN), lambda i: (0, 0)),
        out_shape=jax.ShapeDtypeStruct((512, 2 * HIDDEN), jnp.float32),
    )(table_t)


def _reduce_body(p_ref, o_ref):
    o_ref[...] = jnp.sum(p_ref[...], axis=1)


def _distmult_sc_body(node_hbm, rel_hbm, hlo_hbm, hhi_hbm, rel_idx_hbm,
                      tlo_hbm, thi_hbm, out_hbm, idx_hlo, idx_hhi, idx_r,
                      idx_tlo, idx_thi, hlo_v, hhi_v, r_v, tlo_v, thi_v,
                      pacc_v, sem_h, sem_r, sem_t):
    wid = lax.axis_index("s") * 2 + lax.axis_index("c")

    pltpu.sync_copy(hlo_hbm.at[wid], idx_hlo)
    pltpu.sync_copy(hhi_hbm.at[wid], idx_hhi)
    pltpu.sync_copy(rel_idx_hbm.at[wid], idx_r)
    pltpu.sync_copy(tlo_hbm.at[wid], idx_tlo)
    pltpu.sync_copy(thi_hbm.at[wid], idx_thi)

    def gather(j):
        return (
            pltpu.make_async_copy(node_hbm.at[idx_hlo.at[j]], hlo_v, sem_h),
            pltpu.make_async_copy(node_hbm.at[idx_hhi.at[j]], hhi_v, sem_h),
            pltpu.make_async_copy(rel_hbm.at[idx_r.at[j]], r_v, sem_r),
            pltpu.make_async_copy(node_hbm.at[idx_tlo.at[j]], tlo_v, sem_t),
            pltpu.make_async_copy(node_hbm.at[idx_thi.at[j]], thi_v, sem_t),
        )

    for j in range(N_CHUNKS):
        for c in gather(j):
            c.start()
        for c in gather(j):
            c.wait()

        def row_body(i, _):
            acc = jnp.zeros((LANES,), jnp.float32)
            for g in range(HIDDEN // LANES):
                lo = pl.ds(g * LANES, LANES)
                hi = pl.ds(HIDDEN + g * LANES, LANES)
                hk = hlo_v[i, lo] + hhi_v[i, hi]
                tk = tlo_v[i, lo] + thi_v[i, hi]
                acc = acc + hk * r_v[i, lo] * tk
            pacc_v[i] = acc
            return 0

        lax.fori_loop(0, CHUNK, row_body, 0)
        pltpu.sync_copy(
            pacc_v, out_hbm.at[pl.ds(wid * B_PER_W + j * CHUNK, CHUNK)])


def _split_idx(idx):
    dummy = NODE_ZERO + (jnp.arange(BATCH, dtype=jnp.int32) % NODE_BLK)
    in_lo = idx < NODE_SPLIT
    lo = jnp.where(in_lo, idx, dummy).reshape(N_WORKERS, N_CHUNKS, CHUNK)
    hi = jnp.where(in_lo, dummy, idx - NODE_SPLIT).reshape(
        N_WORKERS, N_CHUNKS, CHUNK)
    return lo, hi


def kernel(head_index, rel_type, tail_index, node_emb, rel_emb):
    hlo, hhi = _split_idx(head_index)
    tlo, thi = _split_idx(tail_index)
    rel3d = rel_type.reshape(N_WORKERS, N_CHUNKS, CHUNK)
    node_tab = _node_table(node_emb.T)
    rel_tab = _rel_table(rel_emb.T)

    mesh = plsc.VectorSubcoreMesh(core_axis_name="c", subcore_axis_name="s")
    idx_t = pltpu.VMEM((N_CHUNKS, CHUNK), jnp.int32)
    row_t = pltpu.VMEM((CHUNK, 2 * HIDDEN), jnp.float32)
    sc_run = functools.partial(
        pl.kernel,
        mesh=mesh,
        compiler_params=pltpu.CompilerParams(use_tc_tiling_on_sc=True),
        out_type=jax.ShapeDtypeStruct((BATCH, LANES), jnp.float32),
        scratch_types=[
            idx_t, idx_t, idx_t, idx_t, idx_t,
            row_t, row_t, row_t, row_t, row_t,
            pltpu.VMEM((CHUNK, LANES), jnp.float32),
            pltpu.SemaphoreType.DMA,
            pltpu.SemaphoreType.DMA,
            pltpu.SemaphoreType.DMA,
        ],
    )(_distmult_sc_body)
    pacc = sc_run(node_tab, rel_tab, hlo, hhi, rel3d, tlo, thi)

    return pl.pallas_call(
        _reduce_body,
        grid=(BATCH // RED_BLK,),
        in_specs=[pl.BlockSpec((RED_BLK, LANES), lambda i: (i, 0))],
        out_specs=pl.BlockSpec((RED_BLK,), lambda i: (i,)),
        out_shape=jax.ShapeDtypeStruct((BATCH,), jnp.float32),
    )(pacc)


# speedup vs baseline: 6.1278x; 1.2474x over previous
import functools

import jax
import jax.numpy as jnp
from jax import lax
from jax.experimental import pallas as pl
from jax.experimental.pallas import tpu as pltpu
from jax.experimental.pallas import tpu_sc as plsc

BATCH = 16384
HIDDEN = 64
LANES = 16
N_CHUNKS = 4
CHUNK = 128
B_PER_W = N_CHUNKS * CHUNK
N_WORKERS = BATCH // B_PER_W
NODE_BLK = 4096
NODE_SPLIT = 122 * NODE_BLK
NODE_DATA = 123 * NODE_BLK
NODE_ZERO = NODE_DATA
RED_BLK = 2048


def _pack_split_body(a_ref, b_ref, dst_ref):
    dst_ref[...] = jnp.concatenate([a_ref[...], b_ref[...]], axis=0).T


def _zero_body(tab_any, dst_ref):
    del tab_any
    dst_ref[...] = jnp.zeros_like(dst_ref)


def _node_table(table_t):
    s_blk = NODE_SPLIT // NODE_BLK
    tab = pl.pallas_call(
        _pack_split_body,
        grid=(NODE_DATA // NODE_BLK,),
        in_specs=[
            pl.BlockSpec((HIDDEN, NODE_BLK), lambda i: (0, i)),
            pl.BlockSpec((HIDDEN, NODE_BLK),
                         lambda i, s_blk=s_blk: (0, i + s_blk)),
        ],
        out_specs=pl.BlockSpec((NODE_BLK, 2 * HIDDEN), lambda i: (i, 0)),
        out_shape=jax.ShapeDtypeStruct((NODE_DATA + NODE_BLK, 2 * HIDDEN),
                                       jnp.float32),
    )(table_t, table_t)
    return pl.pallas_call(
        _zero_body,
        grid=(1,),
        in_specs=[pl.BlockSpec(memory_space=pltpu.MemorySpace.HBM)],
        out_specs=pl.BlockSpec(
            (NODE_BLK, 2 * HIDDEN),
            lambda i: (NODE_DATA // NODE_BLK, 0)),
        out_shape=jax.ShapeDtypeStruct((NODE_DATA + NODE_BLK, 2 * HIDDEN),
                                       jnp.float32),
        input_output_aliases={0: 0},
    )(tab)


def _pack_dup_body(a_ref, dst_ref):
    a = a_ref[...]
    dst_ref[...] = jnp.concatenate([a, a], axis=0).T


def _rel_table(table_t):
    return pl.pallas_call(
        _pack_dup_body,
        grid=(1,),
        in_specs=[pl.BlockSpec((HIDDEN, 512), lambda i: (0, 0))],
        out_specs=pl.BlockSpec((512, 2 * HIDDEN), lambda i: (0, 0)),
        out_shape=jax.ShapeDtypeStruct((512, 2 * HIDDEN), jnp.float32),
    )(table_t)


def _reduce_body(p_ref, o_ref):
    o_ref[...] = jnp.sum(p_ref[...], axis=1)


def _distmult_sc_body(node_hbm, rel_hbm, hlo_hbm, hhi_hbm, rel_idx_hbm,
                      tlo_hbm, thi_hbm, out_hbm, idx_hlo, idx_hhi, idx_r,
                      idx_tlo, idx_thi, hlo_v, hhi_v, r_v, tlo_v, thi_v,
                      pacc_v, sem_h, sem_r, sem_t):
    wid = lax.axis_index("s") * 2 + lax.axis_index("c")

    pltpu.sync_copy(hlo_hbm.at[wid], idx_hlo)
    pltpu.sync_copy(hhi_hbm.at[wid], idx_hhi)
    pltpu.sync_copy(rel_idx_hbm.at[wid], idx_r)
    pltpu.sync_copy(tlo_hbm.at[wid], idx_tlo)
    pltpu.sync_copy(thi_hbm.at[wid], idx_thi)

    def gather(j):
        return (
            pltpu.make_async_copy(node_hbm.at[idx_hlo.at[j]], hlo_v, sem_h),
            pltpu.make_async_copy(node_hbm.at[idx_hhi.at[j]], hhi_v, sem_h),
            pltpu.make_async_copy(rel_hbm.at[idx_r.at[j]], r_v, sem_r),
            pltpu.make_async_copy(node_hbm.at[idx_tlo.at[j]], tlo_v, sem_t),
            pltpu.make_async_copy(node_hbm.at[idx_thi.at[j]], thi_v, sem_t),
        )

    for j in range(N_CHUNKS):
        for c in gather(j):
            c.start()
        for c in gather(j):
            c.wait()

        def row_body(i, _):
            acc = jnp.zeros((LANES,), jnp.float32)
            for g in range(HIDDEN // LANES):
                lo = pl.ds(g * LANES, LANES)
                hi = pl.ds(HIDDEN + g * LANES, LANES)
                hk = hlo_v[i, lo] + hhi_v[i, hi]
                tk = tlo_v[i, lo] + thi_v[i, hi]
                acc = acc + hk * r_v[i, lo] * tk
            pacc_v[i] = acc
            return 0

        lax.fori_loop(0, CHUNK, row_body, 0)
        pltpu.sync_copy(
            pacc_v, out_hbm.at[pl.ds(wid * B_PER_W + j * CHUNK, CHUNK)])


def _split_idx(idx):
    dummy = NODE_ZERO + (jnp.arange(BATCH, dtype=jnp.int32) % NODE_BLK)
    in_lo = idx < NODE_SPLIT
    lo = jnp.where(in_lo, idx, dummy).reshape(N_WORKERS, N_CHUNKS, CHUNK)
    hi = jnp.where(in_lo, dummy, idx - NODE_SPLIT).reshape(
        N_WORKERS, N_CHUNKS, CHUNK)
    return lo, hi


def kernel(head_index, rel_type, tail_index, node_emb, rel_emb):
    hlo, hhi = _split_idx(head_index)
    tlo, thi = _split_idx(tail_index)
    rel3d = rel_type.reshape(N_WORKERS, N_CHUNKS, CHUNK)
    node_tab = _node_table(node_emb.T)
    rel_tab = _rel_table(rel_emb.T)

    mesh = plsc.VectorSubcoreMesh(core_axis_name="c", subcore_axis_name="s")
    idx_t = pltpu.VMEM((N_CHUNKS, CHUNK), jnp.int32)
    row_t = pltpu.VMEM((CHUNK, 2 * HIDDEN), jnp.float32)
    sc_run = functools.partial(
        pl.kernel,
        mesh=mesh,
        compiler_params=pltpu.CompilerParams(use_tc_tiling_on_sc=True),
        out_type=jax.ShapeDtypeStruct((BATCH, LANES), jnp.float32),
        scratch_types=[
            idx_t, idx_t, idx_t, idx_t, idx_t,
            row_t, row_t, row_t, row_t, row_t,
            pltpu.VMEM((CHUNK, LANES), jnp.float32),
            pltpu.SemaphoreType.DMA,
            pltpu.SemaphoreType.DMA,
            pltpu.SemaphoreType.DMA,
        ],
    )(_distmult_sc_body)
    pacc = sc_run(node_tab, rel_tab, hlo, hhi, rel3d, tlo, thi)

    return pl.pallas_call(
        _reduce_body,
        grid=(BATCH // RED_BLK,),
        in_specs=[pl.BlockSpec((RED_BLK, LANES), lambda i: (i, 0))],
        out_specs=pl.BlockSpec((RED_BLK,), lambda i: (i,)),
        out_shape=jax.ShapeDtypeStruct((BATCH,), jnp.float32),
    )(pacc)


# speedup vs baseline: 6.8044x; 1.1104x over previous
import functools

import jax
import jax.numpy as jnp
from jax import lax
from jax.experimental import pallas as pl
from jax.experimental.pallas import tpu as pltpu
from jax.experimental.pallas import tpu_sc as plsc

BATCH = 16384
HIDDEN = 64
LANES = 16
N_CHUNKS = 4
CHUNK = 128
B_PER_W = N_CHUNKS * CHUNK
N_WORKERS = BATCH // B_PER_W
NODE_BLK = 8192
NODE_SPLIT = 61 * NODE_BLK
NODE_DATA = 62 * NODE_BLK
NODE_ZERO = NODE_DATA
RED_BLK = 2048


def _pack_split_body(a_ref, b_ref, dst_ref):
    dst_ref[...] = jnp.concatenate([a_ref[...], b_ref[...]], axis=0).T


def _zero_body(tab_any, dst_ref):
    del tab_any
    dst_ref[...] = jnp.zeros_like(dst_ref)


def _node_table(table_t):
    s_blk = NODE_SPLIT // NODE_BLK
    tab = pl.pallas_call(
        _pack_split_body,
        grid=(NODE_DATA // NODE_BLK,),
        in_specs=[
            pl.BlockSpec((HIDDEN, NODE_BLK), lambda i: (0, i)),
            pl.BlockSpec((HIDDEN, NODE_BLK),
                         lambda i, s_blk=s_blk: (0, i + s_blk)),
        ],
        out_specs=pl.BlockSpec((NODE_BLK, 2 * HIDDEN), lambda i: (i, 0)),
        out_shape=jax.ShapeDtypeStruct((NODE_DATA + NODE_BLK, 2 * HIDDEN),
                                       jnp.float32),
    )(table_t, table_t)
    return pl.pallas_call(
        _zero_body,
        grid=(1,),
        in_specs=[pl.BlockSpec(memory_space=pltpu.MemorySpace.HBM)],
        out_specs=pl.BlockSpec(
            (NODE_BLK, 2 * HIDDEN),
            lambda i: (NODE_DATA // NODE_BLK, 0)),
        out_shape=jax.ShapeDtypeStruct((NODE_DATA + NODE_BLK, 2 * HIDDEN),
                                       jnp.float32),
        input_output_aliases={0: 0},
    )(tab)


def _pack_dup_body(a_ref, dst_ref):
    a = a_ref[...]
    dst_ref[...] = jnp.concatenate([a, a], axis=0).T


def _rel_table(table_t):
    return pl.pallas_call(
        _pack_dup_body,
        grid=(1,),
        in_specs=[pl.BlockSpec((HIDDEN, 512), lambda i: (0, 0))],
        out_specs=pl.BlockSpec((512, 2 * HIDDEN), lambda i: (0, 0)),
        out_shape=jax.ShapeDtypeStruct((512, 2 * HIDDEN), jnp.float32),
    )(table_t)


def _reduce_body(p_ref, o_ref):
    o_ref[...] = jnp.sum(p_ref[...], axis=1)


def _distmult_sc_body(node_hbm, rel_hbm, hlo_hbm, hhi_hbm, rel_idx_hbm,
                      tlo_hbm, thi_hbm, out_hbm, idx_hlo, idx_hhi, idx_r,
                      idx_tlo, idx_thi, hlo_v, hhi_v, r_v, tlo_v, thi_v,
                      pacc_v, sem_h, sem_r, sem_t):
    wid = lax.axis_index("s") * 2 + lax.axis_index("c")

    pltpu.sync_copy(hlo_hbm.at[wid], idx_hlo)
    pltpu.sync_copy(hhi_hbm.at[wid], idx_hhi)
    pltpu.sync_copy(rel_idx_hbm.at[wid], idx_r)
    pltpu.sync_copy(tlo_hbm.at[wid], idx_tlo)
    pltpu.sync_copy(thi_hbm.at[wid], idx_thi)

    def gather(j):
        return (
            pltpu.make_async_copy(node_hbm.at[idx_hlo.at[j]], hlo_v, sem_h),
            pltpu.make_async_copy(node_hbm.at[idx_hhi.at[j]], hhi_v, sem_h),
            pltpu.make_async_copy(rel_hbm.at[idx_r.at[j]], r_v, sem_r),
            pltpu.make_async_copy(node_hbm.at[idx_tlo.at[j]], tlo_v, sem_t),
            pltpu.make_async_copy(node_hbm.at[idx_thi.at[j]], thi_v, sem_t),
        )

    for j in range(N_CHUNKS):
        for c in gather(j):
            c.start()
        for c in gather(j):
            c.wait()

        def row_body(i, _):
            acc = jnp.zeros((LANES,), jnp.float32)
            for g in range(HIDDEN // LANES):
                lo = pl.ds(g * LANES, LANES)
                hi = pl.ds(HIDDEN + g * LANES, LANES)
                hk = hlo_v[i, lo] + hhi_v[i, hi]
                tk = tlo_v[i, lo] + thi_v[i, hi]
                acc = acc + hk * r_v[i, lo] * tk
            pacc_v[i] = acc
            return 0

        lax.fori_loop(0, CHUNK, row_body, 0)
        pltpu.sync_copy(
            pacc_v, out_hbm.at[pl.ds(wid * B_PER_W + j * CHUNK, CHUNK)])


def _split_idx(idx):
    dummy = NODE_ZERO + (jnp.arange(BATCH, dtype=jnp.int32) % NODE_BLK)
    in_lo = idx < NODE_SPLIT
    lo = jnp.where(in_lo, idx, dummy).reshape(N_WORKERS, N_CHUNKS, CHUNK)
    hi = jnp.where(in_lo, dummy, idx - NODE_SPLIT).reshape(
        N_WORKERS, N_CHUNKS, CHUNK)
    return lo, hi


def kernel(head_index, rel_type, tail_index, node_emb, rel_emb):
    hlo, hhi = _split_idx(head_index)
    tlo, thi = _split_idx(tail_index)
    rel3d = rel_type.reshape(N_WORKERS, N_CHUNKS, CHUNK)
    node_tab = _node_table(node_emb.T)
    rel_tab = _rel_table(rel_emb.T)

    mesh = plsc.VectorSubcoreMesh(core_axis_name="c", subcore_axis_name="s")
    idx_t = pltpu.VMEM((N_CHUNKS, CHUNK), jnp.int32)
    row_t = pltpu.VMEM((CHUNK, 2 * HIDDEN), jnp.float32)
    sc_run = functools.partial(
        pl.kernel,
        mesh=mesh,
        compiler_params=pltpu.CompilerParams(use_tc_tiling_on_sc=True),
        out_type=jax.ShapeDtypeStruct((BATCH, LANES), jnp.float32),
        scratch_types=[
            idx_t, idx_t, idx_t, idx_t, idx_t,
            row_t, row_t, row_t, row_t, row_t,
            pltpu.VMEM((CHUNK, LANES), jnp.float32),
            pltpu.SemaphoreType.DMA,
            pltpu.SemaphoreType.DMA,
            pltpu.SemaphoreType.DMA,
        ],
    )(_distmult_sc_body)
    pacc = sc_run(node_tab, rel_tab, hlo, hhi, rel3d, tlo, thi)

    return pl.pallas_call(
        _reduce_body,
        grid=(BATCH // RED_BLK,),
        in_specs=[pl.BlockSpec((RED_BLK, LANES), lambda i: (i, 0))],
        out_specs=pl.BlockSpec((RED_BLK,), lambda i: (i,)),
        out_shape=jax.ShapeDtypeStruct((BATCH,), jnp.float32),
    )(pacc)


# speedup vs baseline: 6.9153x; 1.0163x over previous
import functools

import jax
import jax.numpy as jnp
from jax import lax
from jax.experimental import pallas as pl
from jax.experimental.pallas import tpu as pltpu
from jax.experimental.pallas import tpu_sc as plsc

BATCH = 16384
HIDDEN = 64
LANES = 16
N_CHUNKS = 4
CHUNK = 128
B_PER_W = N_CHUNKS * CHUNK
N_WORKERS = BATCH // B_PER_W
NODE_BLK = 16384
NODE_SPLIT = 31 * NODE_BLK
NODE_DATA = 31 * NODE_BLK
NODE_ZERO = NODE_DATA
RED_BLK = 2048


def _pack_split_body(a_ref, b_ref, dst_ref):
    dst_ref[...] = jnp.concatenate([a_ref[...], b_ref[...]], axis=0).T


def _zero_body(tab_any, dst_ref):
    del tab_any
    dst_ref[...] = jnp.zeros_like(dst_ref)


def _node_table(table_t):
    s_blk = NODE_SPLIT // NODE_BLK
    tab = pl.pallas_call(
        _pack_split_body,
        grid=(NODE_DATA // NODE_BLK,),
        in_specs=[
            pl.BlockSpec((HIDDEN, NODE_BLK), lambda i: (0, i)),
            pl.BlockSpec((HIDDEN, NODE_BLK),
                         lambda i, s_blk=s_blk: (0, i + s_blk)),
        ],
        out_specs=pl.BlockSpec((NODE_BLK, 2 * HIDDEN), lambda i: (i, 0)),
        out_shape=jax.ShapeDtypeStruct((NODE_DATA + NODE_BLK, 2 * HIDDEN),
                                       jnp.float32),
    )(table_t, table_t)
    return pl.pallas_call(
        _zero_body,
        grid=(1,),
        in_specs=[pl.BlockSpec(memory_space=pltpu.MemorySpace.HBM)],
        out_specs=pl.BlockSpec(
            (NODE_BLK, 2 * HIDDEN),
            lambda i: (NODE_DATA // NODE_BLK, 0)),
        out_shape=jax.ShapeDtypeStruct((NODE_DATA + NODE_BLK, 2 * HIDDEN),
                                       jnp.float32),
        input_output_aliases={0: 0},
    )(tab)


def _pack_dup_body(a_ref, dst_ref):
    a = a_ref[...]
    dst_ref[...] = jnp.concatenate([a, a], axis=0).T


def _rel_table(table_t):
    return pl.pallas_call(
        _pack_dup_body,
        grid=(1,),
        in_specs=[pl.BlockSpec((HIDDEN, 512), lambda i: (0, 0))],
        out_specs=pl.BlockSpec((512, 2 * HIDDEN), lambda i: (0, 0)),
        out_shape=jax.ShapeDtypeStruct((512, 2 * HIDDEN), jnp.float32),
    )(table_t)


def _reduce_body(p_ref, o_ref):
    o_ref[...] = jnp.sum(p_ref[...], axis=1)


def _distmult_sc_body(node_hbm, rel_hbm, hlo_hbm, hhi_hbm, rel_idx_hbm,
                      tlo_hbm, thi_hbm, out_hbm, idx_hlo, idx_hhi, idx_r,
                      idx_tlo, idx_thi, hlo_v, hhi_v, r_v, tlo_v, thi_v,
                      pacc_v, sem_h, sem_r, sem_t):
    wid = lax.axis_index("s") * 2 + lax.axis_index("c")

    pltpu.sync_copy(hlo_hbm.at[wid], idx_hlo)
    pltpu.sync_copy(hhi_hbm.at[wid], idx_hhi)
    pltpu.sync_copy(rel_idx_hbm.at[wid], idx_r)
    pltpu.sync_copy(tlo_hbm.at[wid], idx_tlo)
    pltpu.sync_copy(thi_hbm.at[wid], idx_thi)

    def gather(j):
        return (
            pltpu.make_async_copy(node_hbm.at[idx_hlo.at[j]], hlo_v, sem_h),
            pltpu.make_async_copy(node_hbm.at[idx_hhi.at[j]], hhi_v, sem_h),
            pltpu.make_async_copy(rel_hbm.at[idx_r.at[j]], r_v, sem_r),
            pltpu.make_async_copy(node_hbm.at[idx_tlo.at[j]], tlo_v, sem_t),
            pltpu.make_async_copy(node_hbm.at[idx_thi.at[j]], thi_v, sem_t),
        )

    for j in range(N_CHUNKS):
        for c in gather(j):
            c.start()
        for c in gather(j):
            c.wait()

        def row_body(i, _):
            acc = jnp.zeros((LANES,), jnp.float32)
            for g in range(HIDDEN // LANES):
                lo = pl.ds(g * LANES, LANES)
                hi = pl.ds(HIDDEN + g * LANES, LANES)
                hk = hlo_v[i, lo] + hhi_v[i, hi]
                tk = tlo_v[i, lo] + thi_v[i, hi]
                acc = acc + hk * r_v[i, lo] * tk
            pacc_v[i] = acc
            return 0

        lax.fori_loop(0, CHUNK, row_body, 0)
        pltpu.sync_copy(
            pacc_v, out_hbm.at[pl.ds(wid * B_PER_W + j * CHUNK, CHUNK)])


def _split_idx(idx):
    dummy = NODE_ZERO + (jnp.arange(BATCH, dtype=jnp.int32) % NODE_BLK)
    in_lo = idx < NODE_SPLIT
    lo = jnp.where(in_lo, idx, dummy).reshape(N_WORKERS, N_CHUNKS, CHUNK)
    hi = jnp.where(in_lo, dummy, idx - NODE_SPLIT).reshape(
        N_WORKERS, N_CHUNKS, CHUNK)
    return lo, hi


def kernel(head_index, rel_type, tail_index, node_emb, rel_emb):
    hlo, hhi = _split_idx(head_index)
    tlo, thi = _split_idx(tail_index)
    rel3d = rel_type.reshape(N_WORKERS, N_CHUNKS, CHUNK)
    node_tab = _node_table(node_emb.T)
    rel_tab = _rel_table(rel_emb.T)

    mesh = plsc.VectorSubcoreMesh(core_axis_name="c", subcore_axis_name="s")
    idx_t = pltpu.VMEM((N_CHUNKS, CHUNK), jnp.int32)
    row_t = pltpu.VMEM((CHUNK, 2 * HIDDEN), jnp.float32)
    sc_run = functools.partial(
        pl.kernel,
        mesh=mesh,
        compiler_params=pltpu.CompilerParams(use_tc_tiling_on_sc=True),
        out_type=jax.ShapeDtypeStruct((BATCH, LANES), jnp.float32),
        scratch_types=[
            idx_t, idx_t, idx_t, idx_t, idx_t,
            row_t, row_t, row_t, row_t, row_t,
            pltpu.VMEM((CHUNK, LANES), jnp.float32),
            pltpu.SemaphoreType.DMA,
            pltpu.SemaphoreType.DMA,
            pltpu.SemaphoreType.DMA,
        ],
    )(_distmult_sc_body)
    pacc = sc_run(node_tab, rel_tab, hlo, hhi, rel3d, tlo, thi)

    return pl.pallas_call(
        _reduce_body,
        grid=(BATCH // RED_BLK,),
        in_specs=[pl.BlockSpec((RED_BLK, LANES), lambda i: (i, 0))],
        out_specs=pl.BlockSpec((RED_BLK,), lambda i: (i,)),
        out_shape=jax.ShapeDtypeStruct((BATCH,), jnp.float32),
    )(pacc)
